# Initial kernel scaffold; baseline (speedup 1.0000x reference)
import functools

import jax
import jax.numpy as jnp
from jax import lax
from jax.experimental import pallas as pl
from jax.experimental.pallas import tpu as pltpu
from jax.experimental.pallas import tpu_sc as plsc

N = 10000
E = 320000
D = 128
HID = 128
H = 4
G = 64

NC = 2
NS = 16
NW = NC * NS
EPW = E // NW
RPT = N // NS

_MESH = plsc.VectorSubcoreMesh(core_axis_name="c", subcore_axis_name="s")


CSEG = 400
NCHUNK = EPW // CSEG


@functools.partial(
    pl.kernel,
    out_type=jax.ShapeDtypeStruct((NC, N, D), jnp.float32),
    mesh=_MESH,
    scratch_types=[
        pltpu.VMEM((CSEG,), jnp.int32),
        pltpu.VMEM((CSEG,), jnp.int32),
        pltpu.VMEM((CSEG,), jnp.float32),
        pltpu.VMEM((CSEG, D), jnp.float32),
        pltpu.VMEM((125, D), jnp.float32),
        pltpu.VMEM_SHARED((N, D), jnp.float32),
        pltpu.SemaphoreType.DMA,
    ],
)
def _seg_sum_k(x_hbm, src_hbm, dst_hbm, ew_hbm, out_hbm,
               src_v, dst_v, ew_v, rows_v, zbuf, acc_sh, sem):
    c = lax.axis_index("c")
    s = lax.axis_index("s")
    w = c * NS + s

    def _z(r, carry):
        for j in range(D // 16):
            zbuf[r, pl.ds(j * 16, 16)] = jnp.zeros((16,), jnp.float32)
        return carry
    lax.fori_loop(0, 125, _z, 0)
    for k in range(5):
        pltpu.sync_copy(zbuf, acc_sh.at[pl.ds(s * RPT + k * 125, 125)])
    plsc.subcore_barrier()

    base0 = w * EPW

    def _chunk(i, carry):
        base = base0 + i * CSEG
        pltpu.sync_copy(src_hbm.at[pl.ds(base, CSEG)], src_v)
        pltpu.sync_copy(dst_hbm.at[pl.ds(base, CSEG)], dst_v)
        pltpu.sync_copy(ew_hbm.at[pl.ds(base, CSEG)], ew_v)
        pltpu.async_copy(x_hbm.at[src_v], rows_v, sem).wait()

        def _scale(e, carry2):
            wsc = ew_v[e]
            for j in range(D // 16):
                rows_v[e, pl.ds(j * 16, 16)] = rows_v[e, pl.ds(j * 16, 16)] * wsc
            return carry2
        lax.fori_loop(0, CSEG, _scale, 0)
        pltpu.sync_copy(rows_v, acc_sh.at[dst_v], add=True)
        return carry
    lax.fori_loop(0, NCHUNK, _chunk, 0)

    plsc.subcore_barrier()
    pltpu.sync_copy(acc_sh.at[pl.ds(s * RPT, RPT)],
                    out_hbm.at[c, pl.ds(s * RPT, RPT)])


BROWS = 1000


def _mm_relu_body(p0_ref, p1_ref, w_ref, b_ref, o_ref):
    x = p0_ref[...] + p1_ref[...]
    y = jnp.dot(x, w_ref[...], preferred_element_type=jnp.float32)
    o_ref[...] = jnp.maximum(y + b_ref[...], 0.0)


def _mm_relu(p0, p1, W, b):
    return pl.pallas_call(
        _mm_relu_body,
        grid=(N // BROWS,),
        in_specs=[
            pl.BlockSpec((BROWS, D), lambda i: (i, 0)),
            pl.BlockSpec((BROWS, D), lambda i: (i, 0)),
            pl.BlockSpec((D, HID), lambda i: (0, 0)),
            pl.BlockSpec((1, HID), lambda i: (0, 0)),
        ],
        out_specs=pl.BlockSpec((BROWS, HID), lambda i: (i, 0)),
        out_shape=jax.ShapeDtypeStruct((N, HID), jnp.float32),
    )(p0, p1, W, b)


def _tc2_body(q0_ref, q1_ref, w2_ref, b2_ref, w3_ref, wl_ref, wr_ref,
              feat_o, el_o, er_o, k_o, mel_o, mer_o):
    i = pl.program_id(0)
    ng = pl.num_programs(0)
    x = q0_ref[...] + q1_ref[...]
    h2 = jnp.maximum(
        jnp.dot(x, w2_ref[...], preferred_element_type=jnp.float32)
        + b2_ref[...], 0.0)
    feat = jnp.dot(h2, w3_ref[...], preferred_element_type=jnp.float32)
    feat_o[...] = feat
    el = jnp.dot(feat, wl_ref[...], preferred_element_type=jnp.float32)
    er = jnp.dot(feat, wr_ref[...], preferred_element_type=jnp.float32)
    el_o[...] = el
    er_o[...] = er
    bl = jnp.max(el, axis=0, keepdims=True)
    br = jnp.max(er, axis=0, keepdims=True)

    @pl.when(i == 0)
    def _():
        mel_o[...] = bl
        mer_o[...] = br

    @pl.when(i > 0)
    def _():
        mel_o[...] = jnp.maximum(mel_o[...], bl)
        mer_o[...] = jnp.maximum(mer_o[...], br)

    @pl.when(i == ng - 1)
    def _():
        z = mel_o[...] + mer_o[...]
        k_o[...] = jnp.maximum(z, 0.2 * z)


def _tc2(q0, q1, W2, b2, W3, Wl, Wr):
    return pl.pallas_call(
        _tc2_body,
        grid=(N // BROWS,),
        in_specs=[
            pl.BlockSpec((BROWS, HID), lambda i: (i, 0)),
            pl.BlockSpec((BROWS, HID), lambda i: (i, 0)),
            pl.BlockSpec((HID, HID), lambda i: (0, 0)),
            pl.BlockSpec((1, HID), lambda i: (0, 0)),
            pl.BlockSpec((HID, H * HID), lambda i: (0, 0)),
            pl.BlockSpec((H * HID, H), lambda i: (0, 0)),
            pl.BlockSpec((H * HID, H), lambda i: (0, 0)),
        ],
        out_specs=[
            pl.BlockSpec((BROWS, H * HID), lambda i: (i, 0)),
            pl.BlockSpec((BROWS, H), lambda i: (i, 0)),
            pl.BlockSpec((BROWS, H), lambda i: (i, 0)),
            pl.BlockSpec((1, H), lambda i: (0, 0)),
            pl.BlockSpec((1, H), lambda i: (0, 0)),
            pl.BlockSpec((1, H), lambda i: (0, 0)),
        ],
        out_shape=[
            jax.ShapeDtypeStruct((N, H * HID), jnp.float32),
            jax.ShapeDtypeStruct((N, H), jnp.float32),
            jax.ShapeDtypeStruct((N, H), jnp.float32),
            jax.ShapeDtypeStruct((1, H), jnp.float32),
            jax.ShapeDtypeStruct((1, H), jnp.float32),
            jax.ShapeDtypeStruct((1, H), jnp.float32),
        ],
    )(q0, q1, W2, b2, W3, Wl, Wr)


CS3 = 400
EPT3 = E // NS
NCH3 = EPT3 // CS3


@functools.partial(
    pl.kernel,
    out_type=[
        jax.ShapeDtypeStruct((E, H), jnp.float32),
        jax.ShapeDtypeStruct((N, H), jnp.float32),
    ],
    mesh=_MESH,
    scratch_types=[
        pltpu.VMEM((N, H), jnp.float32),
        pltpu.VMEM((N, H), jnp.float32),
        pltpu.VMEM((1, H), jnp.float32),
        pltpu.VMEM((CS3,), jnp.int32),
        pltpu.VMEM((CS3,), jnp.int32),
        pltpu.VMEM((CS3, H), jnp.float32),
        pltpu.VMEM((RPT, H), jnp.float32),
        pltpu.VMEM_SHARED((N, H), jnp.float32),
        pltpu.SemaphoreType.DMA,
    ],
)
def _gat_edge_k(el_hbm, er_hbm, k_hbm, src_hbm, dst_hbm, ee_hbm, den_hbm,
                elv, erv, kv, src_v, dst_v, ee_v, zbuf, den_sh, sem):
    c = lax.axis_index("c")
    s = lax.axis_index("s")

    @pl.when(c == 0)
    def _core0():
        def _z(r, carry):
            for hh in range(H):
                zbuf[r, hh] = 0.0
            return carry
        lax.fori_loop(0, RPT, _z, 0)
        pltpu.sync_copy(zbuf, den_sh.at[pl.ds(s * RPT, RPT)])

    plsc.subcore_barrier()

    @pl.when(c == 0)
    def _core0b():
        pltpu.sync_copy(el_hbm, elv)
        pltpu.sync_copy(er_hbm, erv)
        pltpu.sync_copy(k_hbm, kv)
        lane = jnp.arange(16, dtype=jnp.int32)
        base0 = s * EPT3

        def _chunk(i, carry):
            base = base0 + i * CS3
            pltpu.sync_copy(src_hbm.at[pl.ds(base, CS3)], src_v)
            pltpu.sync_copy(dst_hbm.at[pl.ds(base, CS3)], dst_v)

            def _grp(g, carry2):
                off = g * 16
                s16 = src_v[pl.ds(off, 16)]
                d16 = dst_v[pl.ds(off, 16)]
                row = lane + off
                for hh in range(H):
                    hv = jnp.full((16,), hh, jnp.int32)
                    elh = plsc.load_gather(elv, [s16, hv])
                    erh = plsc.load_gather(erv, [d16, hv])
                    x = elh + erh
                    ex = jnp.maximum(x, 0.2 * x) - kv[0, hh]
                    plsc.store_scatter(ee_v, [row, hv], jnp.exp(ex))
                return carry2
            lax.fori_loop(0, CS3 // 16, _grp, 0)
            pltpu.sync_copy(ee_v, ee_hbm.at[pl.ds(base, CS3)])
            pltpu.sync_copy(ee_v, den_sh.at[dst_v], add=True)
            return carry
        lax.fori_loop(0, NCH3, _chunk, 0)

    plsc.subcore_barrier()

    @pl.when((c == 0) & (s < 10))
    def _read():
        pltpu.sync_copy(den_sh.at[pl.ds(s * 1000, 1000)],
                        den_hbm.at[pl.ds(s * 1000, 1000)])


C4 = 80
NCH4 = EPW // C4


@functools.partial(
    pl.kernel,
    out_type=jax.ShapeDtypeStruct((NC, N, D), jnp.float32),
    mesh=_MESH,
    scratch_types=[
        pltpu.VMEM((N, H), jnp.float32),
        pltpu.VMEM((C4,), jnp.int32),
        pltpu.VMEM((C4,), jnp.int32),
        pltpu.VMEM((C4,), jnp.float32),
        pltpu.VMEM((C4, H), jnp.float32),
        pltpu.VMEM((C4, H), jnp.float32),
        pltpu.VMEM((C4, H * HID), jnp.float32),
        pltpu.VMEM((C4, D), jnp.float32),
        pltpu.VMEM((125, D), jnp.float32),
        pltpu.VMEM_SHARED((N, D), jnp.float32),
        pltpu.SemaphoreType.DMA,
    ],
)
def _gat_agg_k(feat_hbm, src_hbm, dst_hbm, ew_hbm, ee_hbm, den_hbm, out_hbm,
               denv, src_v, dst_v, ew_v, ee_v, al_v, rows_v, ctr_v, zbuf,
               acc_sh, sem):
    c = lax.axis_index("c")
    s = lax.axis_index("s")
    w = c * NS + s

    def _z(r, carry):
        for j in range(D // 16):
            zbuf[r, pl.ds(j * 16, 16)] = jnp.zeros((16,), jnp.float32)
        return carry
    lax.fori_loop(0, 125, _z, 0)
    for k in range(5):
        pltpu.sync_copy(zbuf, acc_sh.at[pl.ds(s * RPT + k * 125, 125)])
    plsc.subcore_barrier()

    pltpu.sync_copy(den_hbm, denv)
    lane = jnp.arange(16, dtype=jnp.int32)
    base0 = w * EPW

    def _chunk(i, carry):
        base = base0 + i * C4
        pltpu.sync_copy(src_hbm.at[pl.ds(base, C4)], src_v)
        pltpu.sync_copy(dst_hbm.at[pl.ds(base, C4)], dst_v)
        pltpu.sync_copy(ew_hbm.at[pl.ds(base, C4)], ew_v)
        pltpu.sync_copy(ee_hbm.at[pl.ds(base, C4)], ee_v)
        pltpu.async_copy(feat_hbm.at[src_v], rows_v, sem).wait()

        for g in range(C4 // 16):
            off = g * 16
            d16 = dst_v[pl.ds(off, 16)]
            ew16 = ew_v[pl.ds(off, 16)]
            row = lane + off
            for hh in range(H):
                hv = jnp.full((16,), hh, jnp.int32)
                eh = plsc.load_gather(ee_v, [row, hv])
                dh = plsc.load_gather(denv, [d16, hv])
                ah = eh / (dh + 1e-16) * ew16
                plsc.store_scatter(al_v, [row, hv], ah)

        def _edge(e, carry2):
            a0 = al_v[e, 0]
            a1 = al_v[e, 1]
            a2 = al_v[e, 2]
            a3 = al_v[e, 3]
            for j in range(D // 16):
                v = (rows_v[e, pl.ds(0 * D + j * 16, 16)] * a0
                     + rows_v[e, pl.ds(1 * D + j * 16, 16)] * a1
                     + rows_v[e, pl.ds(2 * D + j * 16, 16)] * a2
                     + rows_v[e, pl.ds(3 * D + j * 16, 16)] * a3)
                ctr_v[e, pl.ds(j * 16, 16)] = v
            return carry2
        lax.fori_loop(0, C4, _edge, 0)

        pltpu.sync_copy(ctr_v, acc_sh.at[dst_v], add=True)
        return carry
    lax.fori_loop(0, NCH4, _chunk, 0)

    plsc.subcore_barrier()
    pltpu.sync_copy(acc_sh.at[pl.ds(s * RPT, RPT)],
                    out_hbm.at[c, pl.ds(s * RPT, RPT)])


def _tc3_body(r0_ref, r1_ref, b3s_ref, gid_ref, wc1_ref, bc1_ref, g1_ref,
              be1_ref, wc2_ref, bc2_ref, g2_ref, be2_ref, wc3_ref, bc3_ref,
              probs_o, hg_o):
    i = pl.program_id(0)
    ng = pl.num_programs(0)
    hn = jnp.maximum((r0_ref[...] + r1_ref[...] + b3s_ref[...]) / H, 0.0)
    gid = gid_ref[...]
    giota = lax.broadcasted_iota(jnp.float32, (1, G), 1)
    onehot = (gid == giota).astype(jnp.float32)
    part = lax.dot_general(onehot, hn, (((0,), (0,)), ((), ())),
                           preferred_element_type=jnp.float32)

    @pl.when(i == 0)
    def _():
        hg_o[...] = part

    @pl.when(i > 0)
    def _():
        hg_o[...] = hg_o[...] + part

    @pl.when(i == ng - 1)
    def _():
        hg = hg_o[...]

        def bn(x, g, b):
            m = jnp.mean(x, axis=0, keepdims=True)
            v = jnp.mean((x - m) ** 2, axis=0, keepdims=True)
            return (x - m) / jnp.sqrt(v + 1e-5) * g + b

        x1 = jnp.dot(hg, wc1_ref[...], preferred_element_type=jnp.float32)
        x1 = jnp.maximum(bn(x1 + bc1_ref[...], g1_ref[...], be1_ref[...]), 0.0)
        x2 = jnp.dot(x1, wc2_ref[...], preferred_element_type=jnp.float32)
        x2 = jnp.maximum(bn(x2 + bc2_ref[...], g2_ref[...], be2_ref[...]), 0.0)
        lg = jnp.dot(x2, wc3_ref[...], preferred_element_type=jnp.float32)
        lg = lg + bc3_ref[...]
        mx = jnp.max(lg, axis=-1, keepdims=True)
        ex = jnp.exp(lg - mx)
        probs_o[...] = ex / jnp.sum(ex, axis=-1, keepdims=True)


def _tc3(r0, r1, b3sum, gidf, Wc1, bc1, g1, be1, Wc2, bc2, g2, be2, Wc3, bc3):
    probs, _ = pl.pallas_call(
        _tc3_body,
        grid=(N // BROWS,),
        in_specs=[
            pl.BlockSpec((BROWS, D), lambda i: (i, 0)),
            pl.BlockSpec((BROWS, D), lambda i: (i, 0)),
            pl.BlockSpec((1, D), lambda i: (0, 0)),
            pl.BlockSpec((BROWS, 1), lambda i: (i, 0)),
            pl.BlockSpec((HID, 256), lambda i: (0, 0)),
            pl.BlockSpec((1, 256), lambda i: (0, 0)),
            pl.BlockSpec((1, 256), lambda i: (0, 0)),
            pl.BlockSpec((1, 256), lambda i: (0, 0)),
            pl.BlockSpec((256, 128), lambda i: (0, 0)),
            pl.BlockSpec((1, 128), lambda i: (0, 0)),
            pl.BlockSpec((1, 128), lambda i: (0, 0)),
            pl.BlockSpec((1, 128), lambda i: (0, 0)),
            pl.BlockSpec((128, 2), lambda i: (0, 0)),
            pl.BlockSpec((1, 2), lambda i: (0, 0)),
        ],
        out_specs=[
            pl.BlockSpec((G, 2), lambda i: (0, 0)),
            pl.BlockSpec((G, D), lambda i: (0, 0)),
        ],
        out_shape=[
            jax.ShapeDtypeStruct((G, 2), jnp.float32),
            jax.ShapeDtypeStruct((G, D), jnp.float32),
        ],
    )(r0, r1, b3sum, gidf, Wc1, bc1, g1, be1, Wc2, bc2, g2, be2, Wc3, bc3)
    return probs


def kernel(h, edge_index, edge_weight, node_graph_ids, W1, b1, W2, b2, W3, b3,
           attn_l, attn_r, Wc1, bc1, g1, be1, Wc2, bc2, g2, be2, Wc3, bc3):
    src = edge_index[0]
    dst = edge_index[1]

    eye = jnp.eye(H, dtype=jnp.float32)
    Wl = (attn_l[:, :, None] * eye[:, None, :]).reshape(H * HID, H)
    Wr = (attn_r[:, :, None] * eye[:, None, :]).reshape(H * HID, H)
    b3sum = b3.reshape(H, HID).sum(axis=0, keepdims=True)
    gidf = node_graph_ids.astype(jnp.float32).reshape(N, 1)

    p = _seg_sum_k(h, src, dst, edge_weight)
    h1 = _mm_relu(p[0], p[1], W1, b1.reshape(1, HID))
    q = _seg_sum_k(h1, src, dst, edge_weight)
    feat, el, er, K, _, _ = _tc2(q[0], q[1], W2, b2.reshape(1, HID), W3, Wl, Wr)
    ee, den = _gat_edge_k(el, er, K, src, dst)
    Rp = _gat_agg_k(feat, src, dst, edge_weight, ee, den)
    return _tc3(Rp[0], Rp[1], b3sum, gidf, Wc1, bc1.reshape(1, 256),
                g1.reshape(1, 256), be1.reshape(1, 256), Wc2,
                bc2.reshape(1, 128), g2.reshape(1, 128), be2.reshape(1, 128),
                Wc3, bc3.reshape(1, 2))


# speedup vs baseline: 21.7144x; 21.7144x over previous
import functools

import jax
import jax.numpy as jnp
from jax import lax
from jax.experimental import pallas as pl
from jax.experimental.pallas import tpu as pltpu
from jax.experimental.pallas import tpu_sc as plsc

N = 10000
E = 320000
D = 128
HID = 128
H = 4
G = 64

NC = 2
NS = 16
NW = NC * NS
EPW = E // NW
RPT = N // NS
RA = 624
TOFF = NS * RA

_MESH = plsc.VectorSubcoreMesh(core_axis_name="c", subcore_axis_name="s")


CSEG = 256
NSEG_FULL = EPW // CSEG
NCHUNK = EPW // CSEG


@functools.partial(
    pl.kernel,
    out_type=jax.ShapeDtypeStruct((NC, N, D), jnp.float32),
    mesh=_MESH,
    compiler_params=pltpu.CompilerParams(needs_layout_passes=False),
    scratch_types=[
        pltpu.VMEM((CSEG,), jnp.int32),
        pltpu.VMEM((CSEG,), jnp.int32),
        pltpu.VMEM((CSEG,), jnp.float32),
        pltpu.VMEM((CSEG, D), jnp.float32),
        pltpu.VMEM_SHARED((N, D), jnp.float32),
        pltpu.SemaphoreType.DMA,
        [pltpu.SemaphoreType.DMA for _ in range(3)],
    ],
)
def _seg_sum_k(x_hbm, src_hbm, dst_hbm, ew_hbm, zeros_hbm, out_hbm,
               src_v, dst_v, ew_v, rows_v, acc_sh, sem, sems):
    c = lax.axis_index("c")
    s = lax.axis_index("s")
    w = c * NS + s

    pltpu.sync_copy(zeros_hbm.at[pl.ds(s * RA, RA)],
                    acc_sh.at[pl.ds(s * RA, RA)])

    @pl.when(s == NS - 1)
    def _ztail():
        pltpu.sync_copy(zeros_hbm.at[pl.ds(TOFF, N - TOFF)],
                        acc_sh.at[pl.ds(TOFF, N - TOFF)])
    plsc.subcore_barrier()

    base0 = w * EPW

    def _do_chunk(base, zero_head):
        c1 = pltpu.async_copy(src_hbm.at[pl.ds(base, CSEG)], src_v, sems[0])
        c2 = pltpu.async_copy(dst_hbm.at[pl.ds(base, CSEG)], dst_v, sems[1])
        c3 = pltpu.async_copy(ew_hbm.at[pl.ds(base, CSEG)], ew_v, sems[2])
        c1.wait()
        c2.wait()
        c3.wait()
        if zero_head:
            nz = CSEG - (EPW - NSEG_FULL * CSEG)
            for z in range(nz // 16):
                ew_v[pl.ds(z * 16, 16)] = jnp.zeros((16,), jnp.float32)
        pltpu.async_copy(x_hbm.at[src_v], rows_v, sem).wait()

        def _scale(g, carry2):
            off = g * 16
            ew16 = ew_v[pl.ds(off, 16)]
            for k in range(16):
                wsc = ew16[k]
                for j in range(D // 16):
                    rows_v[off + k, pl.ds(j * 16, 16)] = (
                        rows_v[off + k, pl.ds(j * 16, 16)] * wsc)
            return carry2
        lax.fori_loop(0, CSEG // 16, _scale, 0)
        pltpu.sync_copy(rows_v, acc_sh.at[dst_v], add=True)

    def _chunk(i, carry):
        _do_chunk(base0 + i * CSEG, False)
        return carry
    lax.fori_loop(0, NSEG_FULL, _chunk, 0)
    _do_chunk(base0 + EPW - CSEG, True)

    plsc.subcore_barrier()
    pltpu.sync_copy(acc_sh.at[pl.ds(s * RA, RA)],
                    out_hbm.at[c, pl.ds(s * RA, RA)])

    @pl.when(s == NS - 1)
    def _rtail():
        pltpu.sync_copy(acc_sh.at[pl.ds(TOFF, N - TOFF)],
                        out_hbm.at[c, pl.ds(TOFF, N - TOFF)])


BROWS = 1000


def _mm_relu_body(p0_ref, p1_ref, w_ref, b_ref, o_ref):
    x = p0_ref[...] + p1_ref[...]
    y = jnp.dot(x, w_ref[...], preferred_element_type=jnp.float32)
    o_ref[...] = jnp.maximum(y + b_ref[...], 0.0)


def _mm_relu(p0, p1, W, b):
    return pl.pallas_call(
        _mm_relu_body,
        grid=(N // BROWS,),
        in_specs=[
            pl.BlockSpec((BROWS, D), lambda i: (i, 0)),
            pl.BlockSpec((BROWS, D), lambda i: (i, 0)),
            pl.BlockSpec((D, HID), lambda i: (0, 0)),
            pl.BlockSpec((1, HID), lambda i: (0, 0)),
        ],
        out_specs=pl.BlockSpec((BROWS, HID), lambda i: (i, 0)),
        out_shape=jax.ShapeDtypeStruct((N, HID), jnp.float32),
    )(p0, p1, W, b)


def _tc2_body(q0_ref, q1_ref, w2_ref, b2_ref, w3_ref, wl_ref, wr_ref,
              feat_o, el_o, er_o, k_o, mel_o, mer_o):
    i = pl.program_id(0)
    ng = pl.num_programs(0)
    x = q0_ref[...] + q1_ref[...]
    h2 = jnp.maximum(
        jnp.dot(x, w2_ref[...], preferred_element_type=jnp.float32)
        + b2_ref[...], 0.0)
    feat = jnp.dot(h2, w3_ref[...], preferred_element_type=jnp.float32)
    feat_o[...] = feat
    al = wl_ref[...]
    ar = wr_ref[...]
    els = []
    ers = []
    for hh in range(H):
        fh = feat[:, hh * HID:(hh + 1) * HID]
        els.append(jnp.sum(fh * al[hh:hh + 1, :], axis=1, keepdims=True))
        ers.append(jnp.sum(fh * ar[hh:hh + 1, :], axis=1, keepdims=True))
    el = jnp.concatenate(els, axis=1)
    er = jnp.concatenate(ers, axis=1)
    el_o[...] = el
    er_o[...] = er
    bl = jnp.max(el, axis=0, keepdims=True)
    br = jnp.max(er, axis=0, keepdims=True)

    @pl.when(i == 0)
    def _():
        mel_o[...] = bl
        mer_o[...] = br

    @pl.when(i > 0)
    def _():
        mel_o[...] = jnp.maximum(mel_o[...], bl)
        mer_o[...] = jnp.maximum(mer_o[...], br)

    @pl.when(i == ng - 1)
    def _():
        z = mel_o[...] + mer_o[...]
        k_o[...] = jnp.maximum(z, 0.2 * z)


def _tc2(q0, q1, W2, b2, W3, Wl, Wr):
    return pl.pallas_call(
        _tc2_body,
        grid=(N // BROWS,),
        in_specs=[
            pl.BlockSpec((BROWS, HID), lambda i: (i, 0)),
            pl.BlockSpec((BROWS, HID), lambda i: (i, 0)),
            pl.BlockSpec((HID, HID), lambda i: (0, 0)),
            pl.BlockSpec((1, HID), lambda i: (0, 0)),
            pl.BlockSpec((HID, H * HID), lambda i: (0, 0)),
            pl.BlockSpec((8, HID), lambda i: (0, 0)),
            pl.BlockSpec((8, HID), lambda i: (0, 0)),
        ],
        out_specs=[
            pl.BlockSpec((BROWS, H * HID), lambda i: (i, 0)),
            pl.BlockSpec((BROWS, H), lambda i: (i, 0)),
            pl.BlockSpec((BROWS, H), lambda i: (i, 0)),
            pl.BlockSpec((1, H), lambda i: (0, 0)),
            pl.BlockSpec((1, H), lambda i: (0, 0)),
            pl.BlockSpec((1, H), lambda i: (0, 0)),
        ],
        out_shape=[
            jax.ShapeDtypeStruct((N, H * HID), jnp.float32),
            jax.ShapeDtypeStruct((N, H), jnp.float32),
            jax.ShapeDtypeStruct((N, H), jnp.float32),
            jax.ShapeDtypeStruct((1, H), jnp.float32),
            jax.ShapeDtypeStruct((1, H), jnp.float32),
            jax.ShapeDtypeStruct((1, H), jnp.float32),
        ],
    )(q0, q1, W2, b2, W3, Wl, Wr)


CS3 = 2000
EPT3 = E // NS
NCH3 = EPT3 // CS3


@functools.partial(
    pl.kernel,
    out_type=[
        jax.ShapeDtypeStruct((E * H,), jnp.float32),
        jax.ShapeDtypeStruct((E * H,), jnp.float32),
    ],
    mesh=_MESH,
    compiler_params=pltpu.CompilerParams(needs_layout_passes=False),
    scratch_types=[
        pltpu.VMEM((N * H,), jnp.float32),
        pltpu.VMEM((N * H,), jnp.float32),
        pltpu.VMEM((16,), jnp.float32),
        pltpu.VMEM((CS3,), jnp.int32),
        pltpu.VMEM((CS3,), jnp.int32),
        pltpu.VMEM((CS3,), jnp.float32),
        pltpu.VMEM((CS3 * H,), jnp.float32),
        pltpu.VMEM((CS3 * H,), jnp.int32),
        pltpu.VMEM((4000,), jnp.float32),
        pltpu.VMEM_SHARED((N * H,), jnp.float32),
        pltpu.SemaphoreType.DMA,
        [pltpu.SemaphoreType.DMA for _ in range(3)],
    ],
)
def _gat_edge_k(el_hbm, er_hbm, k_hbm, src_hbm, dst_hbm, ew_hbm,
                ee_hbm, al_hbm,
                elv, erdv, kv, src_v, dst_v, ew_v, ee_v, di_v, bb_v,
                den_sh, sem, sems):
    c = lax.axis_index("c")
    s = lax.axis_index("s")

    @pl.when((c == 0) & (s < 10))
    def _zero():
        def _z(r, carry):
            bb_v[pl.ds(r * 16, 16)] = jnp.zeros((16,), jnp.float32)
            return carry
        lax.fori_loop(0, 250, _z, 0)
        pltpu.sync_copy(bb_v, den_sh.at[pl.ds(s * 4000, 4000)])

    plsc.subcore_barrier()

    lane = jnp.arange(16, dtype=jnp.int32)
    base0 = s * EPT3

    @pl.when(c == 0)
    def _phase1():
        pltpu.sync_copy(el_hbm, elv)
        pltpu.sync_copy(er_hbm, erdv)
        pltpu.sync_copy(k_hbm, kv)
        kvec = kv[...]

        def _chunk(i, carry):
            base = base0 + i * CS3
            c1 = pltpu.async_copy(src_hbm.at[pl.ds(base, CS3)], src_v, sems[0])
            c2 = pltpu.async_copy(dst_hbm.at[pl.ds(base, CS3)], dst_v, sems[1])
            c1.wait()
            c2.wait()

            def _grp(g, carry2):
                off = g * 16
                s16 = src_v[pl.ds(off, 16)]
                d16 = dst_v[pl.ds(off, 16)]
                pos0 = (lane + off) * H
                for hh in range(H):
                    elh = plsc.load_gather(elv, [s16 * H + hh])
                    erh = plsc.load_gather(erdv, [d16 * H + hh])
                    x = elh + erh
                    ex = jnp.maximum(x, 0.2 * x) - kvec[hh]
                    plsc.store_scatter(ee_v, [pos0 + hh], jnp.exp(ex))
                    plsc.store_scatter(di_v, [pos0 + hh], d16 * H + hh)
                return carry2
            lax.fori_loop(0, CS3 // 16, _grp, 0)
            pltpu.sync_copy(ee_v, ee_hbm.at[pl.ds(base * H, CS3 * H)])
            pltpu.sync_copy(ee_v, den_sh.at[di_v], add=True)
            return carry
        lax.fori_loop(0, NCH3, _chunk, 0)

    plsc.subcore_barrier()

    @pl.when(c == 0)
    def _phase2():
        pltpu.sync_copy(den_sh, erdv)

        def _chunk2(i, carry):
            base = base0 + i * CS3
            c1 = pltpu.async_copy(dst_hbm.at[pl.ds(base, CS3)], dst_v, sems[0])
            c2 = pltpu.async_copy(ew_hbm.at[pl.ds(base, CS3)], ew_v, sems[1])
            c3 = pltpu.async_copy(ee_hbm.at[pl.ds(base * H, CS3 * H)], ee_v,
                                  sems[2])
            c1.wait()
            c2.wait()
            c3.wait()

            def _grp(g, carry2):
                off = g * 16
                d16 = dst_v[pl.ds(off, 16)]
                ew16 = ew_v[pl.ds(off, 16)]
                pos0 = (lane + off) * H
                for hh in range(H):
                    eh = plsc.load_gather(ee_v, [pos0 + hh])
                    dh = plsc.load_gather(erdv, [d16 * H + hh])
                    ah = eh / (dh + 1e-16) * ew16
                    plsc.store_scatter(ee_v, [pos0 + hh], ah)
                return carry2
            lax.fori_loop(0, CS3 // 16, _grp, 0)
            pltpu.sync_copy(ee_v, al_hbm.at[pl.ds(base * H, CS3 * H)])
            return carry
        lax.fori_loop(0, NCH3, _chunk2, 0)


C4 = 64
NCH4 = 156


@functools.partial(
    pl.kernel,
    out_type=jax.ShapeDtypeStruct((NC, N, D), jnp.float32),
    mesh=_MESH,
    compiler_params=pltpu.CompilerParams(needs_layout_passes=False),
    scratch_types=[
        pltpu.VMEM((C4,), jnp.int32),
        pltpu.VMEM((C4,), jnp.int32),
        pltpu.VMEM((C4 * H,), jnp.float32),
        pltpu.VMEM((C4, H * HID), jnp.float32),
        pltpu.VMEM((C4, D), jnp.float32),
        pltpu.VMEM_SHARED((N, D), jnp.float32),
        pltpu.SemaphoreType.DMA,
        [pltpu.SemaphoreType.DMA for _ in range(3)],
    ],
)
def _gat_agg_k(feat_hbm, src_hbm, dst_hbm, al_hbm, zeros_hbm, out_hbm,
               src_v, dst_v, al_v, rows_v, ctr_v, acc_sh, sem, sems):
    c = lax.axis_index("c")
    s = lax.axis_index("s")
    w = c * NS + s

    pltpu.sync_copy(zeros_hbm.at[pl.ds(s * RA, RA)],
                    acc_sh.at[pl.ds(s * RA, RA)])

    @pl.when(s == NS - 1)
    def _ztail():
        pltpu.sync_copy(zeros_hbm.at[pl.ds(TOFF, N - TOFF)],
                        acc_sh.at[pl.ds(TOFF, N - TOFF)])
    plsc.subcore_barrier()

    lane = jnp.arange(16, dtype=jnp.int32)
    base0 = w * EPW

    def _do_chunk(base, zero_head):
        c1 = pltpu.async_copy(src_hbm.at[pl.ds(base, C4)], src_v, sems[0])
        c2 = pltpu.async_copy(dst_hbm.at[pl.ds(base, C4)], dst_v, sems[1])
        c3 = pltpu.async_copy(al_hbm.at[pl.ds(base * H, C4 * H)], al_v, sems[2])
        c1.wait()
        c2.wait()
        c3.wait()
        if zero_head:
            for z in range(48 * H // 16):
                al_v[pl.ds(z * 16, 16)] = jnp.zeros((16,), jnp.float32)
        pltpu.async_copy(feat_hbm.at[src_v], rows_v, sem).wait()

        def _grp(g, carry2):
            off = g * 16
            row = lane + off
            a0v = plsc.load_gather(al_v, [row * H + 0])
            a1v = plsc.load_gather(al_v, [row * H + 1])
            a2v = plsc.load_gather(al_v, [row * H + 2])
            a3v = plsc.load_gather(al_v, [row * H + 3])
            for k in range(16):
                e = off + k
                for j in range(D // 16):
                    v = (rows_v[e, pl.ds(0 * D + j * 16, 16)] * a0v[k]
                         + rows_v[e, pl.ds(1 * D + j * 16, 16)] * a1v[k]
                         + rows_v[e, pl.ds(2 * D + j * 16, 16)] * a2v[k]
                         + rows_v[e, pl.ds(3 * D + j * 16, 16)] * a3v[k])
                    ctr_v[e, pl.ds(j * 16, 16)] = v
            return carry2
        lax.fori_loop(0, C4 // 16, _grp, 0)

        pltpu.sync_copy(ctr_v, acc_sh.at[dst_v], add=True)

    def _chunk(i, carry):
        _do_chunk(base0 + i * C4, False)
        return carry
    lax.fori_loop(0, NCH4, _chunk, 0)
    _do_chunk(base0 + EPW - C4, True)

    plsc.subcore_barrier()
    pltpu.sync_copy(acc_sh.at[pl.ds(s * RA, RA)],
                    out_hbm.at[c, pl.ds(s * RA, RA)])

    @pl.when(s == NS - 1)
    def _rtail():
        pltpu.sync_copy(acc_sh.at[pl.ds(TOFF, N - TOFF)],
                        out_hbm.at[c, pl.ds(TOFF, N - TOFF)])


def _tc3_body(r0_ref, r1_ref, b3s_ref, gid_ref, wc1_ref, bc1_ref, g1_ref,
              be1_ref, wc2_ref, bc2_ref, g2_ref, be2_ref, wc3_ref, bc3_ref,
              probs_o, hg_o):
    i = pl.program_id(0)
    ng = pl.num_programs(0)
    hn = jnp.maximum((r0_ref[...] + r1_ref[...] + b3s_ref[...]) / H, 0.0)
    gid = gid_ref[...]
    giota = lax.broadcasted_iota(jnp.int32, (1, G), 1).astype(jnp.float32)
    onehot = (gid == giota).astype(jnp.float32)
    part = lax.dot_general(onehot, hn, (((0,), (0,)), ((), ())),
                           preferred_element_type=jnp.float32,
                           precision=lax.Precision.HIGHEST)

    @pl.when(i == 0)
    def _():
        hg_o[...] = part

    @pl.when(i > 0)
    def _():
        hg_o[...] = hg_o[...] + part

    @pl.when(i == ng - 1)
    def _():
        hg = hg_o[...]

        def bn(x, g, b):
            m = jnp.mean(x, axis=0, keepdims=True)
            v = jnp.mean((x - m) ** 2, axis=0, keepdims=True)
            return (x - m) / jnp.sqrt(v + 1e-5) * g + b

        x1 = jnp.dot(hg, wc1_ref[...], preferred_element_type=jnp.float32)
        x1 = jnp.maximum(bn(x1 + bc1_ref[...], g1_ref[...], be1_ref[...]), 0.0)
        x2 = jnp.dot(x1, wc2_ref[...], preferred_element_type=jnp.float32)
        x2 = jnp.maximum(bn(x2 + bc2_ref[...], g2_ref[...], be2_ref[...]), 0.0)
        lg = jnp.dot(x2, wc3_ref[...], preferred_element_type=jnp.float32)
        lg = lg + bc3_ref[...]
        mx = jnp.max(lg, axis=-1, keepdims=True)
        ex = jnp.exp(lg - mx)
        probs_o[...] = ex / jnp.sum(ex, axis=-1, keepdims=True)


def _tc3(r0, r1, b3sum, gidf, Wc1, bc1, g1, be1, Wc2, bc2, g2, be2, Wc3, bc3):
    probs, _ = pl.pallas_call(
        _tc3_body,
        grid=(N // BROWS,),
        in_specs=[
            pl.BlockSpec((BROWS, D), lambda i: (i, 0)),
            pl.BlockSpec((BROWS, D), lambda i: (i, 0)),
            pl.BlockSpec((1, D), lambda i: (0, 0)),
            pl.BlockSpec((BROWS, 1), lambda i: (i, 0)),
            pl.BlockSpec((HID, 256), lambda i: (0, 0)),
            pl.BlockSpec((1, 256), lambda i: (0, 0)),
            pl.BlockSpec((1, 256), lambda i: (0, 0)),
            pl.BlockSpec((1, 256), lambda i: (0, 0)),
            pl.BlockSpec((256, 128), lambda i: (0, 0)),
            pl.BlockSpec((1, 128), lambda i: (0, 0)),
            pl.BlockSpec((1, 128), lambda i: (0, 0)),
            pl.BlockSpec((1, 128), lambda i: (0, 0)),
            pl.BlockSpec((128, 2), lambda i: (0, 0)),
            pl.BlockSpec((1, 2), lambda i: (0, 0)),
        ],
        out_specs=[
            pl.BlockSpec((G, 2), lambda i: (0, 0)),
            pl.BlockSpec((G, D), lambda i: (0, 0)),
        ],
        out_shape=[
            jax.ShapeDtypeStruct((G, 2), jnp.float32),
            jax.ShapeDtypeStruct((G, D), jnp.float32),
        ],
    )(r0, r1, b3sum, gidf, Wc1, bc1, g1, be1, Wc2, bc2, g2, be2, Wc3, bc3)
    return probs


def kernel(h, edge_index, edge_weight, node_graph_ids, W1, b1, W2, b2, W3, b3,
           attn_l, attn_r, Wc1, bc1, g1, be1, Wc2, bc2, g2, be2, Wc3, bc3):
    src = edge_index[0]
    dst = edge_index[1]

    Wl = jnp.pad(attn_l, ((0, 8 - H), (0, 0)))
    Wr = jnp.pad(attn_r, ((0, 8 - H), (0, 0)))
    b3sum = b3.reshape(H, HID).sum(axis=0, keepdims=True)
    gidf = node_graph_ids.astype(jnp.float32).reshape(N, 1)

    zeros_nd = jnp.zeros((N, D), jnp.float32)

    p = _seg_sum_k(h, src, dst, edge_weight, zeros_nd)
    h1 = _mm_relu(p[0], p[1], W1, b1.reshape(1, HID))
    q = _seg_sum_k(h1, src, dst, edge_weight, zeros_nd)
    feat, el, er, K, _, _ = _tc2(q[0], q[1], W2, b2.reshape(1, HID), W3, Wl, Wr)
    K16 = jnp.pad(K.reshape(H), (0, 12))
    _, alpha = _gat_edge_k(el.reshape(N * H), er.reshape(N * H), K16,
                           src, dst, edge_weight)
    Rp = _gat_agg_k(feat, src, dst, alpha, zeros_nd)
    return _tc3(Rp[0], Rp[1], b3sum, gidf, Wc1, bc1.reshape(1, 256),
                g1.reshape(1, 256), be1.reshape(1, 256), Wc2,
                bc2.reshape(1, 128), g2.reshape(1, 128), be2.reshape(1, 128),
                Wc3, bc3.reshape(1, 2))
